# baseline (speedup 1.0000x reference)
import functools

import jax
import jax.numpy as jnp
from jax import lax
from jax.experimental import pallas as pl
from jax.experimental.pallas import tpu as pltpu
from jax.experimental.pallas import tpu_sc as plsc

_N = 10000
_E = 320000
_D = 128
_H = 16
_C = 64
_EPS = 1e-5

_NP = 10240
_PR = _NP * _H // 128
_NR = _N * _H // 128
_ROWS = _NP // 16
_NW = 32
_SUP = 2000
_NSUP = 5
_EW = _SUP * _NSUP

_SELU_ALPHA = 1.6732632423543772
_SELU_SCALE = 1.0507009873554805


def _sc_propagate(args, two_inputs):
    mesh = plsc.VectorSubcoreMesh(core_axis_name="c", subcore_axis_name="s")

    @functools.partial(
        pl.kernel,
        out_type=jax.ShapeDtypeStruct((2, _NP, _H), jnp.float32),
        mesh=mesh,
        scratch_types=[
            pltpu.VMEM((_NSUP, _SUP), jnp.int32),
            pltpu.VMEM((_NSUP, _SUP), jnp.int32),
            pltpu.VMEM((2, _SUP, _H), jnp.float32),
            pltpu.VMEM((_ROWS, _H), jnp.float32),
            pltpu.VMEM((_ROWS, _H), jnp.float32),
            pltpu.VMEM_SHARED((_NP, _H), jnp.float32),
            pltpu.VMEM_SHARED((_NP, _H), jnp.float32),
            pltpu.SemaphoreType.DMA,
            pltpu.SemaphoreType.DMA,
        ],
        compiler_params=pltpu.CompilerParams(use_tc_tiling_on_sc=False),
    )
    def k(*refs):
        if two_inputs:
            (h_hbm, src_hbm, dst_hbm, out_hbm, src_v, dst_v, rows_v,
             hbuf, bbuf, acc, tbl, gsem, ssem) = refs
        else:
            (h_hbm, src_hbm, dst_hbm, out_hbm, src_v, dst_v, rows_v,
             hbuf, bbuf, acc, tbl, gsem, ssem) = refs
        c = lax.axis_index("c")
        s = lax.axis_index("s")
        wid = s * 2 + c
        row0 = s * _ROWS
        e0 = wid * _EW

        cps = []
        for j in range(_NSUP):
            cps.append(pltpu.async_copy(
                src_hbm.at[pl.ds(e0 + j * _SUP, _SUP)], src_v.at[j], gsem))
            cps.append(pltpu.async_copy(
                dst_hbm.at[pl.ds(e0 + j * _SUP, _SUP)], dst_v.at[j], gsem))
        if two_inputs:
            pltpu.async_copy(h_hbm.at[0, pl.ds(row0, _ROWS)], hbuf, ssem)
            pltpu.async_copy(h_hbm.at[1, pl.ds(row0, _ROWS)], bbuf,
                             ssem).wait()
            pltpu.make_async_copy(h_hbm.at[0, pl.ds(row0, _ROWS)], hbuf,
                                  ssem).wait()

            def addrow(i, _):
                hbuf[i, :] = hbuf[i, :] + bbuf[i, :]
                return 0

            lax.fori_loop(0, _ROWS, addrow, 0)
        else:
            pltpu.async_copy(h_hbm.at[pl.ds(row0, _ROWS)], hbuf,
                             ssem).wait()
        pltpu.sync_copy(hbuf, tbl.at[pl.ds(row0, _ROWS)])

        def halfrow(i, _):
            hbuf[i, :] = hbuf[i, :] * 0.5
            return 0

        lax.fori_loop(0, _ROWS, halfrow, 0)
        pltpu.sync_copy(hbuf, acc.at[pl.ds(row0, _ROWS)])
        for cp in cps:
            cp.wait()
        plsc.subcore_barrier()

        g0 = pltpu.async_copy(tbl.at[src_v.at[0]], rows_v.at[0], gsem)
        g0.wait()

        def body(j, _):
            slot = lax.rem(j, 2)
            nxt = lax.rem(j + 1, 2)

            @pl.when(j + 1 < _NSUP)
            def _():
                pltpu.async_copy(tbl.at[src_v.at[j + 1]], rows_v.at[nxt],
                                 gsem)

            pltpu.async_copy(rows_v.at[slot], acc.at[dst_v.at[j]], ssem,
                             add=True).wait()

            @pl.when(j + 1 < _NSUP)
            def _():
                pltpu.make_async_copy(tbl.at[src_v.at[j + 1]],
                                      rows_v.at[nxt], gsem).wait()
            return 0

        lax.fori_loop(0, _NSUP, body, 0)
        plsc.subcore_barrier()
        pltpu.sync_copy(acc.at[pl.ds(row0, _ROWS)],
                        out_hbm.at[c, pl.ds(row0, _ROWS)])

    out = k(*args)
    return out.reshape(2, _PR, 128)


def _sc_propagate_h(h_pk, src_pk, dst_pk):
    return _sc_propagate(
        (h_pk.reshape(_NP, _H), src_pk.reshape(_E), dst_pk.reshape(_E)),
        two_inputs=False)


def _sc_propagate_p(p_pk, src_pk, dst_pk):
    return _sc_propagate(
        (p_pk.reshape(2, _NP, _H), src_pk.reshape(_E), dst_pk.reshape(_E)),
        two_inputs=True)


def _tc_repack_edges(edge_index):
    def body(e_ref, s_ref, d_ref):
        s_ref[...] = e_ref[0].reshape(_E // 128, 128)
        d_ref[...] = e_ref[1].reshape(_E // 128, 128)

    return pl.pallas_call(
        body,
        out_shape=[jax.ShapeDtypeStruct((_E // 128, 128), jnp.int32),
                   jax.ShapeDtypeStruct((_E // 128, 128), jnp.int32)],
    )(edge_index)


def _tc_matmul1(x8, W1bd):
    def body(x_ref, w_ref, o_ref):
        o_ref[...] = jnp.dot(x_ref[...], w_ref[...],
                             preferred_element_type=jnp.float32)

    return pl.pallas_call(
        body,
        out_shape=jax.ShapeDtypeStruct((_PR, 128), jnp.float32),
        grid=(8,),
        in_specs=[pl.BlockSpec((_PR // 8, 1024), lambda i: (i, 0)),
                  pl.BlockSpec((1024, 128), lambda i: (0, 0))],
        out_specs=pl.BlockSpec((_PR // 8, 128), lambda i: (i, 0)),
    )(x8, W1bd)


def _tc_bn_selu(p2, b1t, gt, bt):
    def body(p_ref, b_ref, g_ref, be_ref, o_ref):
        h2 = p_ref[0] + p_ref[1] + b_ref[...]
        rows = lax.broadcasted_iota(jnp.int32, (_PR, 128), 0)
        mask = rows < _NR
        lane = lax.broadcasted_iota(jnp.int32, (128, 128), 0)
        lane_t = lax.broadcasted_iota(jnp.int32, (128, 128), 1)
        fold = jnp.where((lane % _H) == (lane_t % _H), 1.0, 0.0)
        hm = jnp.where(mask, h2, 0.0)
        msum = jnp.sum(hm, axis=0, keepdims=True)
        mean = jnp.dot(msum, fold,
                       preferred_element_type=jnp.float32) / _N
        d = jnp.where(mask, h2 - mean, 0.0)
        vsum = jnp.sum(d * d, axis=0, keepdims=True)
        var = jnp.dot(vsum, fold,
                      preferred_element_type=jnp.float32) / _N
        z = (h2 - mean) * lax.rsqrt(var + _EPS) * g_ref[...] + be_ref[...]
        o_ref[...] = _SELU_SCALE * jnp.where(
            z > 0, z, _SELU_ALPHA * (jnp.exp(z) - 1.0))

    return pl.pallas_call(
        body,
        out_shape=jax.ShapeDtypeStruct((_PR, 128), jnp.float32),
        in_specs=[pl.BlockSpec((2, _PR, 128), lambda: (0, 0, 0)),
                  pl.BlockSpec((1, 128), lambda: (0, 0)),
                  pl.BlockSpec((1, 128), lambda: (0, 0)),
                  pl.BlockSpec((1, 128), lambda: (0, 0))],
        out_specs=pl.BlockSpec((_PR, 128), lambda: (0, 0)),
    )(p2, b1t, gt, bt)


def _tc_head(p3, W2e, b2t):
    blk = _PR // 2

    def body(p_ref, w_ref, b_ref, o_ref):
        t = p_ref[0] + p_ref[1]
        z = jnp.dot(t, w_ref[...],
                    preferred_element_type=jnp.float32) + b_ref[...]
        m = z
        for k in (32, 16, 8, 4, 2, 1):
            m = jnp.maximum(m, pltpu.roll(m, 512 - k, 1))
        li = lax.broadcasted_iota(jnp.int32, (512, 512), 0)
        lj = lax.broadcasted_iota(jnp.int32, (512, 512), 1)
        gsum = jnp.where(li // _C == lj // _C, 1.0, 0.0)
        lane = lax.broadcasted_iota(jnp.int32, (blk, 512), 1)
        msk = jnp.where(lane % _C == 0, m, 0.0)
        mb = jnp.dot(msk, gsum, preferred_element_type=jnp.float32)
        e = jnp.exp(z - mb)
        ssum = jnp.dot(e, gsum, preferred_element_type=jnp.float32)
        o_ref[...] = z - mb - jnp.log(ssum)

    return pl.pallas_call(
        body,
        out_shape=jax.ShapeDtypeStruct((_PR, 8 * _C), jnp.float32),
        grid=(2,),
        in_specs=[pl.BlockSpec((2, blk, 128), lambda i: (0, i, 0)),
                  pl.BlockSpec((128, 8 * _C), lambda i: (0, 0)),
                  pl.BlockSpec((1, 8 * _C), lambda i: (0, 0))],
        out_specs=pl.BlockSpec((blk, 8 * _C), lambda i: (i, 0)),
    )(p3, W2e, b2t)


@jax.jit
def kernel(x, edge_index, W1, b1, gamma, beta, W2, b2):
    src_pk, dst_pk = _tc_repack_edges(edge_index)
    x8 = jnp.pad(x, ((0, _NP - _N), (0, 0))).reshape(_PR, 8 * _D)
    W1bd = jnp.kron(jnp.eye(8, dtype=jnp.float32), W1)
    tile8 = lambda v: jnp.tile(v, 8).reshape(1, 128)
    h0 = _tc_matmul1(x8, W1bd)
    p1 = _sc_propagate_h(h0, src_pk, dst_pk)
    p2 = _sc_propagate_p(p1, src_pk, dst_pk)
    s = _tc_bn_selu(p2, tile8(b1), tile8(gamma), tile8(beta))
    p3 = _sc_propagate_h(s, src_pk, dst_pk)
    W2e = jnp.kron(jnp.eye(8, dtype=jnp.float32), W2)
    b2t = jnp.tile(b2, 8).reshape(1, 8 * _C)
    return _tc_head(p3, W2e, b2t)[:_NR].reshape(_N, _C)


# speedup vs baseline: 1.0765x; 1.0765x over previous
import functools

import jax
import jax.numpy as jnp
from jax import lax
from jax.experimental import pallas as pl
from jax.experimental.pallas import tpu as pltpu
from jax.experimental.pallas import tpu_sc as plsc

_N = 10000
_E = 320000
_D = 128
_H = 16
_C = 64
_EPS = 1e-5

_NP = 10240
_PR = _NP * _H // 128
_NR = _N * _H // 128
_ROWS = _NP // 16
_NW = 32
_SUP = 2000
_NSUP = 5
_EW = _SUP * _NSUP

_SELU_ALPHA = 1.6732632423543772
_SELU_SCALE = 1.0507009873554805


def _sc_scatter_partials(h_pk, src_pk, dst_pk):
    mesh = plsc.VectorSubcoreMesh(core_axis_name="c", subcore_axis_name="s")

    @functools.partial(
        pl.kernel,
        out_type=jax.ShapeDtypeStruct((2, _NP, _H), jnp.float32),
        mesh=mesh,
        scratch_types=[
            pltpu.VMEM((_NSUP, _SUP), jnp.int32),
            pltpu.VMEM((_NSUP, _SUP), jnp.int32),
            pltpu.VMEM((2, _SUP, _H), jnp.float32),
            pltpu.VMEM((_ROWS, _H), jnp.float32),
            pltpu.VMEM_SHARED((_NP, _H), jnp.float32),
            pltpu.VMEM_SHARED((_NP, _H), jnp.float32),
            pltpu.SemaphoreType.DMA,
            pltpu.SemaphoreType.DMA,
        ],
        compiler_params=pltpu.CompilerParams(use_tc_tiling_on_sc=False),
    )
    def k(h_hbm, src_hbm, dst_hbm, out_hbm, src_v, dst_v, rows_v, zbuf,
          acc, tbl, gsem, ssem):
        c = lax.axis_index("c")
        s = lax.axis_index("s")
        wid = s * 2 + c
        row0 = s * _ROWS
        e0 = wid * _EW

        cps = [pltpu.async_copy(h_hbm.at[pl.ds(row0, _ROWS)],
                                tbl.at[pl.ds(row0, _ROWS)], gsem)]
        for j in range(_NSUP):
            cps.append(pltpu.async_copy(
                src_hbm.at[pl.ds(e0 + j * _SUP, _SUP)], src_v.at[j], gsem))
            cps.append(pltpu.async_copy(
                dst_hbm.at[pl.ds(e0 + j * _SUP, _SUP)], dst_v.at[j], gsem))

        def zrow(i, _):
            zbuf[i, :] = jnp.zeros((_H,), jnp.float32)
            return 0

        lax.fori_loop(0, _ROWS, zrow, 0)
        pltpu.sync_copy(zbuf, acc.at[pl.ds(row0, _ROWS)])
        for cp in cps:
            cp.wait()
        plsc.subcore_barrier()

        g0 = pltpu.async_copy(tbl.at[src_v.at[0]], rows_v.at[0], gsem)
        g0.wait()

        def body(j, _):
            slot = lax.rem(j, 2)
            nxt = lax.rem(j + 1, 2)

            @pl.when(j + 1 < _NSUP)
            def _():
                pltpu.async_copy(tbl.at[src_v.at[j + 1]], rows_v.at[nxt],
                                 gsem)

            pltpu.async_copy(rows_v.at[slot], acc.at[dst_v.at[j]], ssem,
                             add=True).wait()

            @pl.when(j + 1 < _NSUP)
            def _():
                pltpu.make_async_copy(tbl.at[src_v.at[j + 1]],
                                      rows_v.at[nxt], gsem).wait()
            return 0

        lax.fori_loop(0, _NSUP, body, 0)
        plsc.subcore_barrier()
        pltpu.sync_copy(acc.at[pl.ds(row0, _ROWS)],
                        out_hbm.at[c, pl.ds(row0, _ROWS)])

    out = k(h_pk.reshape(_NP, _H), src_pk.reshape(_E), dst_pk.reshape(_E))
    return out.reshape(2, _PR, 128)


def _tc_repack_edges(edge_index):
    def body(e_ref, s_ref, d_ref):
        s_ref[...] = e_ref[0].reshape(_E // 128, 128)
        d_ref[...] = e_ref[1].reshape(_E // 128, 128)

    return pl.pallas_call(
        body,
        out_shape=[jax.ShapeDtypeStruct((_E // 128, 128), jnp.int32),
                   jax.ShapeDtypeStruct((_E // 128, 128), jnp.int32)],
    )(edge_index)


def _tc_matmul1(x8, W1bd):
    def body(x_ref, w_ref, o_ref):
        o_ref[...] = jnp.dot(x_ref[...], w_ref[...],
                             preferred_element_type=jnp.float32)

    return pl.pallas_call(
        body,
        out_shape=jax.ShapeDtypeStruct((_PR, 128), jnp.float32),
        grid=(8,),
        in_specs=[pl.BlockSpec((_PR // 8, 1024), lambda i: (i, 0)),
                  pl.BlockSpec((1024, 128), lambda i: (0, 0))],
        out_specs=pl.BlockSpec((_PR // 8, 128), lambda i: (i, 0)),
    )(x8, W1bd)


def _tc_combine(h, p):
    def body(h_ref, p_ref, o_ref):
        o_ref[...] = h_ref[...] + p_ref[0] + p_ref[1]

    return pl.pallas_call(
        body,
        out_shape=jax.ShapeDtypeStruct((_PR, 128), jnp.float32),
        grid=(5,),
        in_specs=[pl.BlockSpec((_PR // 5, 128), lambda i: (i, 0)),
                  pl.BlockSpec((2, _PR // 5, 128), lambda i: (0, i, 0))],
        out_specs=pl.BlockSpec((_PR // 5, 128), lambda i: (i, 0)),
    )(h, p)


def _tc_bn_selu(h1, p2, b1t, gt, bt):
    def body(h_ref, p_ref, b_ref, g_ref, be_ref, o_ref):
        h2 = h_ref[...] + p_ref[0] + p_ref[1] + b_ref[...]
        rows = lax.broadcasted_iota(jnp.int32, (_PR, 128), 0)
        mask = rows < _NR
        lane = lax.broadcasted_iota(jnp.int32, (128, 128), 0)
        lane_t = lax.broadcasted_iota(jnp.int32, (128, 128), 1)
        fold = jnp.where((lane % _H) == (lane_t % _H), 1.0, 0.0)
        hm = jnp.where(mask, h2, 0.0)
        msum = jnp.sum(hm, axis=0, keepdims=True)
        mean = jnp.dot(msum, fold,
                       preferred_element_type=jnp.float32) / _N
        d = jnp.where(mask, h2 - mean, 0.0)
        vsum = jnp.sum(d * d, axis=0, keepdims=True)
        var = jnp.dot(vsum, fold,
                      preferred_element_type=jnp.float32) / _N
        z = (h2 - mean) * lax.rsqrt(var + _EPS) * g_ref[...] + be_ref[...]
        o_ref[...] = _SELU_SCALE * jnp.where(
            z > 0, z, _SELU_ALPHA * (jnp.exp(z) - 1.0))

    return pl.pallas_call(
        body,
        out_shape=jax.ShapeDtypeStruct((_PR, 128), jnp.float32),
        in_specs=[pl.BlockSpec((_PR, 128), lambda: (0, 0)),
                  pl.BlockSpec((2, _PR, 128), lambda: (0, 0, 0)),
                  pl.BlockSpec((1, 128), lambda: (0, 0)),
                  pl.BlockSpec((1, 128), lambda: (0, 0)),
                  pl.BlockSpec((1, 128), lambda: (0, 0))],
        out_specs=pl.BlockSpec((_PR, 128), lambda: (0, 0)),
    )(h1, p2, b1t, gt, bt)


def _tc_head(s, p3, W2e, b2t):
    blk = _PR // 2

    def body(s_ref, p_ref, w_ref, b_ref, o_ref):
        t = s_ref[...] + p_ref[0] + p_ref[1]
        z = jnp.dot(t, w_ref[...],
                    preferred_element_type=jnp.float32) + b_ref[...]
        m = z
        for k in (32, 16, 8, 4, 2, 1):
            m = jnp.maximum(m, pltpu.roll(m, 512 - k, 1))
        li = lax.broadcasted_iota(jnp.int32, (512, 512), 0)
        lj = lax.broadcasted_iota(jnp.int32, (512, 512), 1)
        gsum = jnp.where(li // _C == lj // _C, 1.0, 0.0)
        lane = lax.broadcasted_iota(jnp.int32, (blk, 512), 1)
        msk = jnp.where(lane % _C == 0, m, 0.0)
        mb = jnp.dot(msk, gsum, preferred_element_type=jnp.float32)
        e = jnp.exp(z - mb)
        ssum = jnp.dot(e, gsum, preferred_element_type=jnp.float32)
        o_ref[...] = z - mb - jnp.log(ssum)

    return pl.pallas_call(
        body,
        out_shape=jax.ShapeDtypeStruct((_PR, 8 * _C), jnp.float32),
        grid=(2,),
        in_specs=[pl.BlockSpec((blk, 128), lambda i: (i, 0)),
                  pl.BlockSpec((2, blk, 128), lambda i: (0, i, 0)),
                  pl.BlockSpec((128, 8 * _C), lambda i: (0, 0)),
                  pl.BlockSpec((1, 8 * _C), lambda i: (0, 0))],
        out_specs=pl.BlockSpec((blk, 8 * _C), lambda i: (i, 0)),
    )(s, p3, W2e, b2t)


@jax.jit
def kernel(x, edge_index, W1, b1, gamma, beta, W2, b2):
    src_pk, dst_pk = _tc_repack_edges(edge_index)
    x8 = jnp.pad(x, ((0, _NP - _N), (0, 0))).reshape(_PR, 8 * _D)
    W1bd = jnp.kron(jnp.eye(8, dtype=jnp.float32), W1)
    tile8 = lambda v: jnp.tile(v, 8).reshape(1, 128)
    h0 = _tc_matmul1(x8, W1bd)
    p1 = _sc_scatter_partials(h0, src_pk, dst_pk)
    h1 = _tc_combine(h0, p1)
    p2 = _sc_scatter_partials(h1, src_pk, dst_pk)
    s = _tc_bn_selu(h1, p2, tile8(b1), tile8(gamma), tile8(beta))
    p3 = _sc_scatter_partials(s, src_pk, dst_pk)
    W2e = jnp.kron(jnp.eye(8, dtype=jnp.float32), W2)
    b2t = jnp.tile(b2, 8).reshape(1, 8 * _C)
    return _tc_head(s, p3, W2e, b2t)[:_NR].reshape(_N, _C)
